# baseline (device time: 912637 ns/iter reference)
import jax
import jax.numpy as jnp
from jax import lax
from jax.experimental import pallas as pl
from jax.experimental.pallas import tpu as pltpu

N_DEV = 8
SQ = 1024
SKV = 1024
HQ = 8
DH = 128
D_MODEL = 1024
SCALE = 0.08838834764831843
BLK = 64


def kernel(x, Wq, K_ext, V_ext, Wo):
    i = lax.axis_index("i")
    x2 = x.reshape(SQ, D_MODEL)
    Kf = K_ext.reshape(SKV, 64 * DH)
    Vf = V_ext.reshape(SKV, 64 * DH)
    Ks = lax.dynamic_slice_in_dim(Kf, i * HQ * DH, HQ * DH, axis=1)
    Vs = lax.dynamic_slice_in_dim(Vf, i * HQ * DH, HQ * DH, axis=1)

    def body(x_ref, wq_ref, k_ref, v_ref, wo_ref, out_ref,
             comm_ref, send_sems, recv_sems):
        my = lax.axis_index("i")
        left = lax.rem(my + N_DEV - 1, N_DEV)
        right = lax.rem(my + 1, N_DEV)

        barrier_sem = pltpu.get_barrier_semaphore()
        for nbr in [left, right]:
            pl.semaphore_signal(
                barrier_sem, inc=1,
                device_id=(nbr,), device_id_type=pl.DeviceIdType.MESH,
            )
        pl.semaphore_wait(barrier_sem, 2)

        q = jnp.dot(x_ref[...], wq_ref[...],
                    preferred_element_type=jnp.float32)

        qb = lax.broadcasted_iota(jnp.int32, (SQ, SKV), 0) // BLK
        kb = lax.broadcasted_iota(jnp.int32, (SQ, SKV), 1) // BLK
        mask = (qb == kb) | (kb == 0) | (lax.rem(qb + kb, 3) == 0)

        acc = jnp.zeros((SQ, D_MODEL), jnp.float32)
        for h in range(HQ):
            qh = q[:, h * DH:(h + 1) * DH]
            kh = k_ref[:, h * DH:(h + 1) * DH]
            vh = v_ref[:, h * DH:(h + 1) * DH]
            s = lax.dot_general(
                qh, kh, (((1,), (1,)), ((), ())),
                preferred_element_type=jnp.float32) * SCALE
            s = jnp.where(mask, s, -1e9)
            m = jnp.max(s, axis=1, keepdims=True)
            w = jnp.exp(s - m)
            w = w / jnp.sum(w, axis=1, keepdims=True)
            ctx_h = jnp.dot(w, vh, preferred_element_type=jnp.float32)
            acc = acc + jnp.dot(
                ctx_h, wo_ref[h * DH:(h + 1) * DH, :],
                preferred_element_type=jnp.float32)

        out_ref[...] = acc
        comm_ref[0, :, :] = acc

        for h in range(N_DEV - 1):
            send_slot = h % 2
            recv_slot = (h + 1) % 2
            rdma = pltpu.make_async_remote_copy(
                src_ref=comm_ref.at[send_slot],
                dst_ref=comm_ref.at[recv_slot],
                send_sem=send_sems.at[send_slot],
                recv_sem=recv_sems.at[recv_slot],
                device_id=(right,),
                device_id_type=pl.DeviceIdType.MESH,
            )
            rdma.start()
            rdma.wait()
            out_ref[...] = out_ref[...] + comm_ref[recv_slot, :, :]

    out = pl.pallas_call(
        body,
        out_shape=jax.ShapeDtypeStruct((SQ, D_MODEL), jnp.float32),
        in_specs=[pl.BlockSpec(memory_space=pltpu.VMEM)] * 5,
        out_specs=pl.BlockSpec(memory_space=pltpu.VMEM),
        scratch_shapes=[
            pltpu.VMEM((2, SQ, D_MODEL), jnp.float32),
            pltpu.SemaphoreType.DMA((2,)),
            pltpu.SemaphoreType.DMA((2,)),
        ],
        compiler_params=pltpu.CompilerParams(collective_id=0),
    )(x2, Wq, Ks, Vs, Wo)
    return out.reshape(1, SQ, D_MODEL)


# device time: 376457 ns/iter; 2.4243x vs baseline; 2.4243x over previous
import jax
import jax.numpy as jnp
from jax import lax
from jax.experimental import pallas as pl
from jax.experimental.pallas import tpu as pltpu

N_DEV = 8
SQ = 1024
SKV = 1024
HQ = 8
DH = 128
D_MODEL = 1024
SCALE = 0.08838834764831843
BLK = 64


def kernel(x, Wq, K_ext, V_ext, Wo):
    x2 = x.reshape(SQ, D_MODEL)

    def body(x_ref, wq_ref, k_hbm, v_hbm, wo_ref, out_ref,
             kv_ref, comm_ref, kv_sems, send_sems, recv_sems):
        my = lax.axis_index("i")
        left = lax.rem(my + N_DEV - 1, N_DEV)
        right = lax.rem(my + 1, N_DEV)

        kcopy = pltpu.make_async_copy(
            k_hbm.at[0, :, pl.ds(my * HQ, HQ), :], kv_ref.at[0], kv_sems.at[0])
        vcopy = pltpu.make_async_copy(
            v_hbm.at[0, :, pl.ds(my * HQ, HQ), :], kv_ref.at[1], kv_sems.at[1])
        kcopy.start()
        vcopy.start()

        q = jnp.dot(x_ref[...], wq_ref[...],
                    preferred_element_type=jnp.float32)

        qb = lax.broadcasted_iota(jnp.int32, (SQ, SKV), 0) // BLK
        kb = lax.broadcasted_iota(jnp.int32, (SQ, SKV), 1) // BLK
        mask = (qb == kb) | (kb == 0) | (lax.rem(qb + kb, 3) == 0)

        kcopy.wait()
        vcopy.wait()

        acc = jnp.zeros((SQ, D_MODEL), jnp.float32)
        for h in range(HQ):
            qh = q[:, h * DH:(h + 1) * DH]
            kh = kv_ref[0, :, h, :]
            vh = kv_ref[1, :, h, :]
            s = lax.dot_general(
                qh, kh, (((1,), (1,)), ((), ())),
                preferred_element_type=jnp.float32) * SCALE
            s = jnp.where(mask, s, -1e9)
            m = jnp.max(s, axis=1, keepdims=True)
            w = jnp.exp(s - m)
            w = w / jnp.sum(w, axis=1, keepdims=True)
            ctx_h = jnp.dot(w, vh, preferred_element_type=jnp.float32)
            acc = acc + jnp.dot(
                ctx_h, wo_ref[h * DH:(h + 1) * DH, :],
                preferred_element_type=jnp.float32)

        out_ref[...] = acc
        comm_ref[0, :, :] = acc

        barrier_sem = pltpu.get_barrier_semaphore()
        for nbr in [left, right]:
            pl.semaphore_signal(
                barrier_sem, inc=1,
                device_id=(nbr,), device_id_type=pl.DeviceIdType.MESH,
            )
        pl.semaphore_wait(barrier_sem, 2)

        for h in range(N_DEV - 1):
            send_slot = h % 2
            recv_slot = (h + 1) % 2
            rdma = pltpu.make_async_remote_copy(
                src_ref=comm_ref.at[send_slot],
                dst_ref=comm_ref.at[recv_slot],
                send_sem=send_sems.at[send_slot],
                recv_sem=recv_sems.at[recv_slot],
                device_id=(right,),
                device_id_type=pl.DeviceIdType.MESH,
            )
            rdma.start()
            rdma.wait()
            out_ref[...] = out_ref[...] + comm_ref[recv_slot, :, :]

    out = pl.pallas_call(
        body,
        out_shape=jax.ShapeDtypeStruct((SQ, D_MODEL), jnp.float32),
        in_specs=[
            pl.BlockSpec(memory_space=pltpu.VMEM),
            pl.BlockSpec(memory_space=pltpu.VMEM),
            pl.BlockSpec(memory_space=pltpu.MemorySpace.HBM),
            pl.BlockSpec(memory_space=pltpu.MemorySpace.HBM),
            pl.BlockSpec(memory_space=pltpu.VMEM),
        ],
        out_specs=pl.BlockSpec(memory_space=pltpu.VMEM),
        scratch_shapes=[
            pltpu.VMEM((2, SKV, HQ, DH), jnp.float32),
            pltpu.VMEM((2, SQ, D_MODEL), jnp.float32),
            pltpu.SemaphoreType.DMA((2,)),
            pltpu.SemaphoreType.DMA((2,)),
            pltpu.SemaphoreType.DMA((2,)),
        ],
        compiler_params=pltpu.CompilerParams(collective_id=0),
    )(x2, Wq, K_ext, V_ext, Wo)
    return out.reshape(1, SQ, D_MODEL)


# device time: 96130 ns/iter; 9.4938x vs baseline; 3.9161x over previous
import jax
import jax.numpy as jnp
from jax import lax
from jax.experimental import pallas as pl
from jax.experimental.pallas import tpu as pltpu

N_DEV = 8
SQ = 1024
SKV = 1024
HQ = 8
DH = 128
D_MODEL = 1024
SCALE = 0.08838834764831843
BLK = 64

RX, RY, RZ, AZ, AY, AX, RTOT = 0, 512, 768, 896, 1024, 1280, 1792


def kernel(x, Wq, K_ext, V_ext, Wo):
    x2 = x.reshape(SQ, D_MODEL)

    def body(x_ref, wq_ref, k_hbm, v_hbm, wo_ref, out_ref,
             kv_ref, sbuf, rbuf, kv_sems, ssems, rsems):
        my = lax.axis_index("i")
        dz = my // 4
        p = lax.rem(my, 4)
        dy = p // 2
        dx = lax.rem(p + dy, 2)

        def pos(ax, ay, az):
            return az * 4 + ay * 2 + lax.rem(ax + ay, 2)

        xpart = pos(1 - dx, dy, dz)
        ypart = pos(dx, 1 - dy, dz)
        zpart = lax.rem(my + 4, N_DEV)

        kcopy = pltpu.make_async_copy(
            k_hbm.at[0, :, pl.ds(my * HQ, HQ), :], kv_ref.at[0], kv_sems.at[0])
        vcopy = pltpu.make_async_copy(
            v_hbm.at[0, :, pl.ds(my * HQ, HQ), :], kv_ref.at[1], kv_sems.at[1])
        kcopy.start()
        vcopy.start()

        q = jnp.dot(x_ref[...], wq_ref[...],
                    preferred_element_type=jnp.float32)

        qb = lax.broadcasted_iota(jnp.int32, (SQ, SKV), 0) // BLK
        kb = lax.broadcasted_iota(jnp.int32, (SQ, SKV), 1) // BLK
        mask = (qb == kb) | (kb == 0) | (lax.rem(qb + kb, 3) == 0)

        kcopy.wait()
        vcopy.wait()

        acc = jnp.zeros((SQ, D_MODEL), jnp.float32)
        for h in range(HQ):
            qh = q[:, h * DH:(h + 1) * DH]
            kh = kv_ref[0, :, h, :]
            vh = kv_ref[1, :, h, :]
            s = lax.dot_general(
                qh, kh, (((1,), (1,)), ((), ())),
                preferred_element_type=jnp.float32) * SCALE
            s = jnp.where(mask, s, -1e9)
            m = jnp.max(s, axis=1, keepdims=True)
            w = jnp.exp(s - m)
            w = w / jnp.sum(w, axis=1, keepdims=True)
            ctx_h = jnp.dot(w, vh, preferred_element_type=jnp.float32)
            acc = acc + jnp.dot(
                ctx_h, wo_ref[h * DH:(h + 1) * DH, :],
                preferred_element_type=jnp.float32)

        out_ref[...] = acc

        barrier_sem = pltpu.get_barrier_semaphore()
        for nbr in [xpart, ypart, zpart]:
            pl.semaphore_signal(
                barrier_sem, inc=1,
                device_id=(nbr,), device_id_type=pl.DeviceIdType.MESH,
            )
        pl.semaphore_wait(barrier_sem, 3)

        kx = 512 * dx
        sx = 512 - kx
        ky = kx + 256 * dy
        sy = kx + 256 - 256 * dy
        kz = ky + 128 * dz
        sz = ky + 128 - 128 * dz

        def exchange(step, nrows, src_start, dst_region, partner):
            sbuf[pl.ds(0, nrows), :] = (
                out_ref[pl.ds(src_start, nrows), :].astype(jnp.bfloat16))
            rdma = pltpu.make_async_remote_copy(
                src_ref=sbuf.at[pl.ds(0, nrows)],
                dst_ref=rbuf.at[pl.ds(dst_region, nrows)],
                send_sem=ssems.at[step],
                recv_sem=rsems.at[step],
                device_id=(partner,),
                device_id_type=pl.DeviceIdType.MESH,
            )
            rdma.start()
            rdma.wait()

        def add_rows(dst_start, region, nrows):
            out_ref[pl.ds(dst_start, nrows), :] = (
                out_ref[pl.ds(dst_start, nrows), :]
                + rbuf[pl.ds(region, nrows), :].astype(jnp.float32))

        def store_rows(dst_start, region, nrows):
            out_ref[pl.ds(dst_start, nrows), :] = (
                rbuf[pl.ds(region, nrows), :].astype(jnp.float32))

        exchange(0, 512, sx, RX, xpart)
        add_rows(kx, RX, 512)
        exchange(1, 256, sy, RY, ypart)
        add_rows(ky, RY, 256)
        exchange(2, 128, sz, RZ, zpart)
        add_rows(kz, RZ, 128)
        exchange(3, 128, kz, AZ, zpart)
        store_rows(sz, AZ, 128)
        exchange(4, 256, ky, AY, ypart)
        store_rows(sy, AY, 256)
        exchange(5, 512, kx, AX, xpart)
        store_rows(sx, AX, 512)

    out = pl.pallas_call(
        body,
        out_shape=jax.ShapeDtypeStruct((SQ, D_MODEL), jnp.float32),
        in_specs=[
            pl.BlockSpec(memory_space=pltpu.VMEM),
            pl.BlockSpec(memory_space=pltpu.VMEM),
            pl.BlockSpec(memory_space=pltpu.MemorySpace.HBM),
            pl.BlockSpec(memory_space=pltpu.MemorySpace.HBM),
            pl.BlockSpec(memory_space=pltpu.VMEM),
        ],
        out_specs=pl.BlockSpec(memory_space=pltpu.VMEM),
        scratch_shapes=[
            pltpu.VMEM((2, SKV, HQ, DH), jnp.float32),
            pltpu.VMEM((512, D_MODEL), jnp.bfloat16),
            pltpu.VMEM((RTOT, D_MODEL), jnp.bfloat16),
            pltpu.SemaphoreType.DMA((2,)),
            pltpu.SemaphoreType.DMA((6,)),
            pltpu.SemaphoreType.DMA((6,)),
        ],
        compiler_params=pltpu.CompilerParams(collective_id=0),
    )(x2, Wq, K_ext, V_ext, Wo)
    return out.reshape(1, SQ, D_MODEL)


# device time: 95466 ns/iter; 9.5598x vs baseline; 1.0070x over previous
import jax
import jax.numpy as jnp
from jax import lax
from jax.experimental import pallas as pl
from jax.experimental.pallas import tpu as pltpu

N_DEV = 8
SQ = 1024
SKV = 1024
HQ = 8
DH = 128
D_MODEL = 1024
SCALE = 0.08838834764831843
BLK = 64

RX, RY, RZ, AZ, AY, AX, RTOT = 0, 512, 768, 896, 1024, 1280, 1792


def kernel(x, Wq, K_ext, V_ext, Wo):
    x2 = x.reshape(SQ, D_MODEL)

    def body(x_ref, wq_ref, k_hbm, v_hbm, wo_ref, out_ref,
             kv_ref, sbuf, rbuf, kv_sems, ssems, rsems):
        my = lax.axis_index("i")
        dz = my // 4
        p = lax.rem(my, 4)
        dy = p // 2
        dx = lax.rem(p + dy, 2)

        def pos(ax, ay, az):
            return az * 4 + ay * 2 + lax.rem(ax + ay, 2)

        xpart = pos(1 - dx, dy, dz)
        ypart = pos(dx, 1 - dy, dz)
        zpart = lax.rem(my + 4, N_DEV)

        kcopy = pltpu.make_async_copy(
            k_hbm.at[0, :, pl.ds(my * HQ, HQ), :], kv_ref.at[0], kv_sems.at[0])
        vcopy = pltpu.make_async_copy(
            v_hbm.at[0, :, pl.ds(my * HQ, HQ), :], kv_ref.at[1], kv_sems.at[1])
        kcopy.start()
        vcopy.start()

        q = jnp.dot(x_ref[...].astype(jnp.bfloat16),
                    wq_ref[...].astype(jnp.bfloat16),
                    preferred_element_type=jnp.float32)

        qb = lax.broadcasted_iota(jnp.int32, (SQ, SKV), 0) // BLK
        kb = lax.broadcasted_iota(jnp.int32, (SQ, SKV), 1) // BLK
        mask = (qb == kb) | (kb == 0) | (lax.rem(qb + kb, 3) == 0)

        kcopy.wait()
        vcopy.wait()

        acc = jnp.zeros((SQ, D_MODEL), jnp.float32)
        for h in range(HQ):
            qh = q[:, h * DH:(h + 1) * DH].astype(jnp.bfloat16)
            kh = kv_ref[0, :, h, :].astype(jnp.bfloat16)
            vh = kv_ref[1, :, h, :].astype(jnp.bfloat16)
            s = lax.dot_general(
                qh, kh, (((1,), (1,)), ((), ())),
                preferred_element_type=jnp.float32) * SCALE
            w = jnp.exp(jnp.where(mask, s, -1e9))
            w = w / jnp.sum(w, axis=1, keepdims=True)
            ctx_h = jnp.dot(w.astype(jnp.bfloat16), vh,
                            preferred_element_type=jnp.float32)
            acc = acc + jnp.dot(
                ctx_h.astype(jnp.bfloat16),
                wo_ref[h * DH:(h + 1) * DH, :].astype(jnp.bfloat16),
                preferred_element_type=jnp.float32)

        out_ref[...] = acc

        barrier_sem = pltpu.get_barrier_semaphore()
        for nbr in [xpart, ypart, zpart]:
            pl.semaphore_signal(
                barrier_sem, inc=1,
                device_id=(nbr,), device_id_type=pl.DeviceIdType.MESH,
            )
        pl.semaphore_wait(barrier_sem, 3)

        kx = 512 * dx
        sx = 512 - kx
        ky = kx + 256 * dy
        sy = kx + 256 - 256 * dy
        kz = ky + 128 * dz
        sz = ky + 128 - 128 * dz

        def exchange(step, nrows, src_start, dst_region, partner):
            sbuf[pl.ds(0, nrows), :] = (
                out_ref[pl.ds(src_start, nrows), :].astype(jnp.bfloat16))
            rdma = pltpu.make_async_remote_copy(
                src_ref=sbuf.at[pl.ds(0, nrows)],
                dst_ref=rbuf.at[pl.ds(dst_region, nrows)],
                send_sem=ssems.at[step],
                recv_sem=rsems.at[step],
                device_id=(partner,),
                device_id_type=pl.DeviceIdType.MESH,
            )
            rdma.start()
            rdma.wait()

        def add_rows(dst_start, region, nrows):
            out_ref[pl.ds(dst_start, nrows), :] = (
                out_ref[pl.ds(dst_start, nrows), :]
                + rbuf[pl.ds(region, nrows), :].astype(jnp.float32))

        def store_rows(dst_start, region, nrows):
            out_ref[pl.ds(dst_start, nrows), :] = (
                rbuf[pl.ds(region, nrows), :].astype(jnp.float32))

        exchange(0, 512, sx, RX, xpart)
        add_rows(kx, RX, 512)
        exchange(1, 256, sy, RY, ypart)
        add_rows(ky, RY, 256)
        exchange(2, 128, sz, RZ, zpart)
        add_rows(kz, RZ, 128)
        exchange(3, 128, kz, AZ, zpart)
        store_rows(sz, AZ, 128)
        exchange(4, 256, ky, AY, ypart)
        store_rows(sy, AY, 256)
        exchange(5, 512, kx, AX, xpart)
        store_rows(sx, AX, 512)

    out = pl.pallas_call(
        body,
        out_shape=jax.ShapeDtypeStruct((SQ, D_MODEL), jnp.float32),
        in_specs=[
            pl.BlockSpec(memory_space=pltpu.VMEM),
            pl.BlockSpec(memory_space=pltpu.VMEM),
            pl.BlockSpec(memory_space=pltpu.MemorySpace.HBM),
            pl.BlockSpec(memory_space=pltpu.MemorySpace.HBM),
            pl.BlockSpec(memory_space=pltpu.VMEM),
        ],
        out_specs=pl.BlockSpec(memory_space=pltpu.VMEM),
        scratch_shapes=[
            pltpu.VMEM((2, SKV, HQ, DH), jnp.float32),
            pltpu.VMEM((512, D_MODEL), jnp.bfloat16),
            pltpu.VMEM((RTOT, D_MODEL), jnp.bfloat16),
            pltpu.SemaphoreType.DMA((2,)),
            pltpu.SemaphoreType.DMA((6,)),
            pltpu.SemaphoreType.DMA((6,)),
        ],
        compiler_params=pltpu.CompilerParams(collective_id=0),
    )(x2, Wq, K_ext, V_ext, Wo)
    return out.reshape(1, SQ, D_MODEL)


# device time: 88058 ns/iter; 10.3640x vs baseline; 1.0841x over previous
import jax
import jax.numpy as jnp
from jax import lax
from jax.experimental import pallas as pl
from jax.experimental.pallas import tpu as pltpu

N_DEV = 8
SQ = 1024
SKV = 1024
HQ = 8
DH = 128
D_MODEL = 1024
SCALE = 0.08838834764831843
BLK = 64
HALF = SQ // 2

RX, RY, RZ, AZ, AY, AX, RTOT = 0, 512, 768, 896, 1024, 1280, 1792


def kernel(x, Wq, K_ext, V_ext, Wo):
    x2 = x.reshape(SQ, D_MODEL)

    def body(x_ref, wq_ref, k_hbm, v_hbm, wo_ref, out_ref,
             kv_ref, sbuf, rbuf, kv_sems, ssems, rsems):
        my = lax.axis_index("i")
        dz = my // 4
        p = lax.rem(my, 4)
        dy = p // 2
        dx = lax.rem(p + dy, 2)

        def pos(ax, ay, az):
            return az * 4 + ay * 2 + lax.rem(ax + ay, 2)

        xpart = pos(1 - dx, dy, dz)
        ypart = pos(dx, 1 - dy, dz)
        zpart = lax.rem(my + 4, N_DEV)

        barrier_sem = pltpu.get_barrier_semaphore()
        for nbr in [xpart, ypart, zpart]:
            pl.semaphore_signal(
                barrier_sem, inc=1,
                device_id=(nbr,), device_id_type=pl.DeviceIdType.MESH,
            )
        pl.semaphore_wait(barrier_sem, 3)

        kcopy = pltpu.make_async_copy(
            k_hbm.at[0, :, pl.ds(my * HQ, HQ), :], kv_ref.at[0], kv_sems.at[0])
        vcopy = pltpu.make_async_copy(
            v_hbm.at[0, :, pl.ds(my * HQ, HQ), :], kv_ref.at[1], kv_sems.at[1])
        kcopy.start()
        vcopy.start()

        kx = 512 * dx
        sx = 512 - kx
        ky = kx + 256 * dy
        sy = kx + 256 - 256 * dy
        kz = ky + 128 * dz
        sz = ky + 128 - 128 * dz

        wqb = (wq_ref[...] * SCALE).astype(jnp.bfloat16)
        kcopy.wait()
        vcopy.wait()

        def compute_rows(row_start):
            xr = x_ref[pl.ds(row_start, HALF), :].astype(jnp.bfloat16)
            q = jnp.dot(xr, wqb, preferred_element_type=jnp.float32)
            qb = (lax.broadcasted_iota(jnp.int32, (HALF, SKV), 0)
                  + row_start) // BLK
            kb = lax.broadcasted_iota(jnp.int32, (HALF, SKV), 1) // BLK
            mask = (qb == kb) | (kb == 0) | (lax.rem(qb + kb, 3) == 0)
            acc = jnp.zeros((HALF, D_MODEL), jnp.float32)
            for h in range(HQ):
                qh = q[:, h * DH:(h + 1) * DH].astype(jnp.bfloat16)
                kh = kv_ref[0, :, h, :].astype(jnp.bfloat16)
                vh = kv_ref[1, :, h, :].astype(jnp.bfloat16)
                s = lax.dot_general(
                    qh, kh, (((1,), (1,)), ((), ())),
                    preferred_element_type=jnp.float32)
                w = jnp.exp(jnp.where(mask, s, -1e9))
                inv = 1.0 / jnp.sum(w, axis=1, keepdims=True)
                ctx = jnp.dot(w.astype(jnp.bfloat16), vh,
                              preferred_element_type=jnp.float32) * inv
                acc = acc + jnp.dot(
                    ctx.astype(jnp.bfloat16),
                    wo_ref[h * DH:(h + 1) * DH, :].astype(jnp.bfloat16),
                    preferred_element_type=jnp.float32)
            return acc

        def make_exchange(step, nrows, dst_region, partner):
            return pltpu.make_async_remote_copy(
                src_ref=sbuf.at[pl.ds(0, nrows)],
                dst_ref=rbuf.at[pl.ds(dst_region, nrows)],
                send_sem=ssems.at[step],
                recv_sem=rsems.at[step],
                device_id=(partner,),
                device_id_type=pl.DeviceIdType.MESH,
            )

        def exchange(step, nrows, src_start, dst_region, partner):
            sbuf[pl.ds(0, nrows), :] = (
                out_ref[pl.ds(src_start, nrows), :].astype(jnp.bfloat16))
            rdma = make_exchange(step, nrows, dst_region, partner)
            rdma.start()
            rdma.wait()

        def add_rows(dst_start, region, nrows):
            out_ref[pl.ds(dst_start, nrows), :] = (
                out_ref[pl.ds(dst_start, nrows), :]
                + rbuf[pl.ds(region, nrows), :].astype(jnp.float32))

        def store_rows(dst_start, region, nrows):
            out_ref[pl.ds(dst_start, nrows), :] = (
                rbuf[pl.ds(region, nrows), :].astype(jnp.float32))

        acc1 = compute_rows(sx)
        out_ref[pl.ds(sx, HALF), :] = acc1
        sbuf[pl.ds(0, HALF), :] = acc1.astype(jnp.bfloat16)
        rs_x = make_exchange(0, 512, RX, xpart)
        rs_x.start()

        acc2 = compute_rows(kx)
        out_ref[pl.ds(kx, HALF), :] = acc2

        rs_x.wait_recv()
        add_rows(kx, RX, 512)
        rs_x.wait_send()
        exchange(1, 256, sy, RY, ypart)
        add_rows(ky, RY, 256)
        exchange(2, 128, sz, RZ, zpart)
        add_rows(kz, RZ, 128)
        exchange(3, 128, kz, AZ, zpart)
        store_rows(sz, AZ, 128)
        exchange(4, 256, ky, AY, ypart)
        store_rows(sy, AY, 256)
        exchange(5, 512, kx, AX, xpart)
        store_rows(sx, AX, 512)

    out = pl.pallas_call(
        body,
        out_shape=jax.ShapeDtypeStruct((SQ, D_MODEL), jnp.float32),
        in_specs=[
            pl.BlockSpec(memory_space=pltpu.VMEM),
            pl.BlockSpec(memory_space=pltpu.VMEM),
            pl.BlockSpec(memory_space=pltpu.MemorySpace.HBM),
            pl.BlockSpec(memory_space=pltpu.MemorySpace.HBM),
            pl.BlockSpec(memory_space=pltpu.VMEM),
        ],
        out_specs=pl.BlockSpec(memory_space=pltpu.VMEM),
        scratch_shapes=[
            pltpu.VMEM((2, SKV, HQ, DH), jnp.float32),
            pltpu.VMEM((512, D_MODEL), jnp.bfloat16),
            pltpu.VMEM((RTOT, D_MODEL), jnp.bfloat16),
            pltpu.SemaphoreType.DMA((2,)),
            pltpu.SemaphoreType.DMA((6,)),
            pltpu.SemaphoreType.DMA((6,)),
        ],
        compiler_params=pltpu.CompilerParams(collective_id=0),
    )(x2, Wq, K_ext, V_ext, Wo)
    return out.reshape(1, SQ, D_MODEL)
